# baseline (speedup 1.0000x reference)
import functools

import jax
import jax.numpy as jnp
from jax import lax
from jax.experimental import pallas as pl
from jax.experimental.pallas import tpu as pltpu
from jax.experimental.pallas import tpu_sc as plsc

N, D = 16384, 64

_info = plsc.get_sparse_core_info()
NC, NS, L = _info.num_cores, _info.num_subcores, _info.num_lanes
NW = NC * NS
ROWS = N // NW

_mesh = plsc.VectorSubcoreMesh(core_axis_name="c", subcore_axis_name="s")


@functools.partial(
    pl.kernel,
    out_type=jax.ShapeDtypeStruct((N,), jnp.float32),
    mesh=_mesh,
    scratch_types=[
        pltpu.VMEM((ROWS // 2, D), jnp.float32),
        pltpu.VMEM((ROWS // 2, D), jnp.float32),
        pltpu.VMEM((ROWS,), jnp.float32),
    ],
)
def _rowdot(gu_hbm, gi_hbm, out_hbm, u_v, i_v, o_v):
    wid = lax.axis_index("s") * NC + lax.axis_index("c")
    base = wid * ROWS

    lanes = lax.iota(jnp.int32, L)
    dnums = lax.GatherDimensionNumbers(
        offset_dims=(), collapsed_slice_dims=(0,), start_index_map=(0,))

    def permute(v, idx):
        return lax.gather(v, idx[:, None], dnums, (1,),
                          mode=lax.GatherScatterMode.PROMISE_IN_BOUNDS)

    def lane_sum(v):
        for sh in (8, 4, 2, 1):
            v = v + permute(v, lanes ^ sh)
        return v

    half = ROWS // 2
    for c in range(2):
        pltpu.sync_copy(gu_hbm.at[pl.ds(base + c * half, half), :], u_v)
        pltpu.sync_copy(gi_hbm.at[pl.ds(base + c * half, half), :], i_v)

        def grp_body(g, carry):
            def lane_body(l, res):
                r = g * L + l
                acc = u_v[r, pl.ds(0, L)] * i_v[r, pl.ds(0, L)]
                for j in range(1, D // L):
                    acc = acc + u_v[r, pl.ds(j * L, L)] * i_v[r, pl.ds(j * L, L)]
                return jnp.where(lanes == l, lane_sum(acc), res)

            res = lax.fori_loop(0, L, lane_body, jnp.zeros((L,), jnp.float32))
            o_v[pl.ds(c * half + g * L, L)] = res
            return carry

        lax.fori_loop(0, half // L, grp_body, 0)
    pltpu.sync_copy(o_v, out_hbm.at[pl.ds(base, ROWS)])


def kernel(gu, gi):
    return _rowdot(gu, gi)
